# baseline (device time: 29556 ns/iter reference)
import jax
import jax.numpy as jnp
from jax import lax
from jax.experimental import pallas as pl
from jax.experimental.pallas import tpu as pltpu

N_DEV = 4
N_LAYERS = 3
WCHUNKS = 4
CCHUNKS = 4
FROM_LEFT, FROM_RIGHT, FROM_DIAG = 0, 1, 2


def kernel(x, Win0, Wout0, Win1, Wout1, Win2, Wout2):
    b, d = x.shape
    out_rows = b // N_DEV
    dc = d // CCHUNKS

    def body(x_ref, win0, wout0, win1, wout1, win2, wout2,
             out_ref, partial_ref, comm_ref, comm3_ref, win_buf, wout_buf,
             send_sems, recv_sems, win_dma_sems, wout_dma_sems):
        my = lax.axis_index("i")
        left = lax.rem(my + N_DEV - 1, N_DEV)
        right = lax.rem(my + 1, N_DEV)
        diag = lax.rem(my + 2, N_DEV)
        peers = ((left, FROM_RIGHT), (right, FROM_LEFT), (diag, FROM_DIAG))

        wins = [win0, win1, win2]
        wouts = [wout0, wout1, wout2]

        def start_weight_dma(l):
            win_copies, wout_copies = [], []
            for c in range(WCHUNKS):
                rw = pl.ds(c * (win_buf.shape[1] // WCHUNKS),
                           win_buf.shape[1] // WCHUNKS)
                cw = pltpu.make_async_copy(
                    wins[l].at[rw], win_buf.at[l % 2, rw],
                    win_dma_sems.at[l % 2, c])
                ro = pl.ds(c * (wout_buf.shape[1] // WCHUNKS),
                           wout_buf.shape[1] // WCHUNKS)
                co = pltpu.make_async_copy(
                    wouts[l].at[ro], wout_buf.at[l % 2, ro],
                    wout_dma_sems.at[l % 2, c])
                win_copies.append(cw)
                wout_copies.append(co)
            for cw in win_copies:
                cw.start()
            for co in wout_copies:
                co.start()
            return win_copies, wout_copies

        pending = start_weight_dma(0)

        barrier = pltpu.get_barrier_semaphore()
        for nbr, _ in peers:
            pl.semaphore_signal(
                barrier, inc=1,
                device_id=(nbr,), device_id_type=pl.DeviceIdType.MESH,
            )
        pl.semaphore_wait(barrier, 3)

        xb = x_ref[...].astype(jnp.bfloat16)
        for l in range(N_LAYERS):
            win_copies, wout_copies = pending
            if l + 1 < N_LAYERS:
                pending = start_weight_dma(l + 1)

            for cw in win_copies:
                cw.wait()
            h = jnp.dot(xb, win_buf[l % 2].astype(jnp.bfloat16),
                        preferred_element_type=jnp.float32)
            h = jnp.maximum(h, 0.0).astype(jnp.bfloat16)
            for co in wout_copies:
                co.wait()

            rdmas = []
            part_cols = []
            if l < N_LAYERS - 1:
                for c in range(CCHUNKS):
                    cols = pl.ds(c * dc, dc)
                    pc = jnp.dot(
                        h, wout_buf[l % 2, :, c * dc:(c + 1) * dc]
                        .astype(jnp.bfloat16),
                        preferred_element_type=jnp.float32)
                    part_cols.append(pc)
                    partial_ref[:, cols] = pc.astype(jnp.bfloat16)
                    for j, (peer, slot) in enumerate(peers):
                        r = pltpu.make_async_remote_copy(
                            src_ref=partial_ref.at[:, cols],
                            dst_ref=comm_ref.at[l, slot, :, cols],
                            send_sem=send_sems.at[l, c, j],
                            recv_sem=recv_sems.at[l, c, slot],
                            device_id=(peer,),
                            device_id_type=pl.DeviceIdType.MESH,
                        )
                        r.start()
                        rdmas.append(r)

                for r in rdmas:
                    r.wait_recv()
                part = jnp.concatenate(part_cols, axis=1)
                total = (part
                         + comm_ref[l, FROM_LEFT].astype(jnp.float32)
                         + comm_ref[l, FROM_RIGHT].astype(jnp.float32)
                         + comm_ref[l, FROM_DIAG].astype(jnp.float32))
                xb = total.astype(jnp.bfloat16)
            else:
                for c in range(CCHUNKS):
                    cols = pl.ds(c * dc, dc)
                    pc = jnp.dot(
                        h, wout_buf[l % 2, :, c * dc:(c + 1) * dc]
                        .astype(jnp.bfloat16),
                        preferred_element_type=jnp.float32)
                    partial_ref[:, cols] = pc.astype(jnp.bfloat16)
                    for j, (peer, slot) in enumerate(peers):
                        r = pltpu.make_async_remote_copy(
                            src_ref=partial_ref.at[
                                pl.ds(peer * out_rows, out_rows), cols],
                            dst_ref=comm3_ref.at[slot, :, cols],
                            send_sem=send_sems.at[l, c, j],
                            recv_sem=recv_sems.at[l, c, slot],
                            device_id=(peer,),
                            device_id_type=pl.DeviceIdType.MESH,
                        )
                        r.start()
                        rdmas.append(r)
                for r in rdmas:
                    r.wait_recv()
                rows = pl.ds(my * out_rows, out_rows)
                out_ref[...] = (
                    partial_ref[rows, :].astype(jnp.float32)
                    + comm3_ref[FROM_LEFT].astype(jnp.float32)
                    + comm3_ref[FROM_RIGHT].astype(jnp.float32)
                    + comm3_ref[FROM_DIAG].astype(jnp.float32))

            for r in rdmas:
                r.wait_send()

    d_in, h_in = Win0.shape
    return pl.pallas_call(
        body,
        out_shape=jax.ShapeDtypeStruct((out_rows, d), jnp.float32),
        in_specs=[pl.BlockSpec(memory_space=pltpu.VMEM)]
        + [pl.BlockSpec(memory_space=pl.ANY)] * 6,
        out_specs=pl.BlockSpec(memory_space=pltpu.VMEM),
        scratch_shapes=[
            pltpu.VMEM((b, d), jnp.bfloat16),
            pltpu.VMEM((N_LAYERS - 1, 3, b, d), jnp.bfloat16),
            pltpu.VMEM((3, out_rows, d), jnp.bfloat16),
            pltpu.VMEM((2, d_in, h_in), jnp.float32),
            pltpu.VMEM((2, h_in, d_in), jnp.float32),
            pltpu.SemaphoreType.DMA((N_LAYERS, CCHUNKS, 3)),
            pltpu.SemaphoreType.DMA((N_LAYERS, CCHUNKS, 3)),
            pltpu.SemaphoreType.DMA((2, WCHUNKS)),
            pltpu.SemaphoreType.DMA((2, WCHUNKS)),
        ],
        compiler_params=pltpu.CompilerParams(
            collective_id=0,
            vmem_limit_bytes=100 * 1024 * 1024,
        ),
    )(x, Win0, Wout0, Win1, Wout1, Win2, Wout2)


# device time: 29344 ns/iter; 1.0072x vs baseline; 1.0072x over previous
import jax
import jax.numpy as jnp
from jax import lax
from jax.experimental import pallas as pl
from jax.experimental.pallas import tpu as pltpu

N_DEV = 4
N_LAYERS = 3
WCHUNKS = 4
CCHUNKS = 2
FROM_LEFT, FROM_RIGHT, FROM_DIAG = 0, 1, 2


def kernel(x, Win0, Wout0, Win1, Wout1, Win2, Wout2):
    b, d = x.shape
    out_rows = b // N_DEV
    dc = d // CCHUNKS

    def body(x_ref, win0, wout0, win1, wout1, win2, wout2,
             out_ref, partial_ref, comm_ref, comm3_ref, win_buf, wout_buf,
             win_bf, wout_bf, send_sems, recv_sems,
             win_dma_sems, wout_dma_sems):
        my = lax.axis_index("i")
        left = lax.rem(my + N_DEV - 1, N_DEV)
        right = lax.rem(my + 1, N_DEV)
        diag = lax.rem(my + 2, N_DEV)
        peers = ((left, FROM_RIGHT), (right, FROM_LEFT), (diag, FROM_DIAG))

        wins = [win0, win1, win2]
        wouts = [wout0, wout1, wout2]

        def start_weight_dma(l):
            win_copies, wout_copies = [], []
            for c in range(WCHUNKS):
                rw = pl.ds(c * (win_buf.shape[1] // WCHUNKS),
                           win_buf.shape[1] // WCHUNKS)
                cw = pltpu.make_async_copy(
                    wins[l].at[rw], win_buf.at[l % 2, rw],
                    win_dma_sems.at[l % 2, c])
                ro = pl.ds(c * (wout_buf.shape[1] // WCHUNKS),
                           wout_buf.shape[1] // WCHUNKS)
                co = pltpu.make_async_copy(
                    wouts[l].at[ro], wout_buf.at[l % 2, ro],
                    wout_dma_sems.at[l % 2, c])
                win_copies.append(cw)
                wout_copies.append(co)
            for cw in win_copies:
                cw.start()
            for co in wout_copies:
                co.start()
            return win_copies, wout_copies

        def wait_and_convert(l, copies):
            win_copies, wout_copies = copies
            for cw in win_copies:
                cw.wait()
            win_bf[l % 2] = win_buf[l % 2].astype(jnp.bfloat16)
            for co in wout_copies:
                co.wait()
            wout_bf[l % 2] = wout_buf[l % 2].astype(jnp.bfloat16)

        pend = {0: start_weight_dma(0), 1: start_weight_dma(1)}

        barrier = pltpu.get_barrier_semaphore()
        for nbr, _ in peers:
            pl.semaphore_signal(
                barrier, inc=1,
                device_id=(nbr,), device_id_type=pl.DeviceIdType.MESH,
            )
        pl.semaphore_wait(barrier, 3)

        wait_and_convert(0, pend.pop(0))
        xb = x_ref[...].astype(jnp.bfloat16)
        for l in range(N_LAYERS):
            if l + 2 < N_LAYERS:
                pend[l + 2] = start_weight_dma(l + 2)

            h = jnp.dot(xb, win_bf[l % 2],
                        preferred_element_type=jnp.float32)
            h = jnp.maximum(h, 0.0).astype(jnp.bfloat16)

            rdmas = []
            part_cols = []
            for c in range(CCHUNKS):
                cols = pl.ds(c * dc, dc)
                pc = jnp.dot(h, wout_bf[l % 2][:, c * dc:(c + 1) * dc],
                             preferred_element_type=jnp.float32)
                part_cols.append(pc)
                partial_ref[:, cols] = pc.astype(jnp.bfloat16)
                chunk_rdmas = []
                for j, (peer, slot) in enumerate(peers):
                    if l < N_LAYERS - 1:
                        src = partial_ref.at[:, cols]
                        dst = comm_ref.at[l, slot, :, cols]
                    else:
                        src = partial_ref.at[
                            pl.ds(peer * out_rows, out_rows), cols]
                        dst = comm3_ref.at[slot, :, cols]
                    r = pltpu.make_async_remote_copy(
                        src_ref=src, dst_ref=dst,
                        send_sem=send_sems.at[l, c, j],
                        recv_sem=recv_sems.at[l, c, slot],
                        device_id=(peer,),
                        device_id_type=pl.DeviceIdType.MESH,
                    )
                    r.start()
                    chunk_rdmas.append(r)
                rdmas.append(chunk_rdmas)

            if l + 1 < N_LAYERS:
                wait_and_convert(l + 1, pend.pop(l + 1))

            if l < N_LAYERS - 1:
                xb_cols = []
                for c in range(CCHUNKS):
                    for r in rdmas[c]:
                        r.wait_recv()
                    cols = slice(c * dc, (c + 1) * dc)
                    tot = (part_cols[c]
                           + comm_ref[l, FROM_LEFT, :, cols]
                           .astype(jnp.float32)
                           + comm_ref[l, FROM_RIGHT, :, cols]
                           .astype(jnp.float32)
                           + comm_ref[l, FROM_DIAG, :, cols]
                           .astype(jnp.float32))
                    xb_cols.append(tot.astype(jnp.bfloat16))
                xb = jnp.concatenate(xb_cols, axis=1)
            else:
                for chunk_rdmas in rdmas:
                    for r in chunk_rdmas:
                        r.wait_recv()
                rows = pl.ds(my * out_rows, out_rows)
                out_ref[...] = (
                    partial_ref[rows, :].astype(jnp.float32)
                    + comm3_ref[FROM_LEFT].astype(jnp.float32)
                    + comm3_ref[FROM_RIGHT].astype(jnp.float32)
                    + comm3_ref[FROM_DIAG].astype(jnp.float32))

            for chunk_rdmas in rdmas:
                for r in chunk_rdmas:
                    r.wait_send()

    d_in, h_in = Win0.shape
    return pl.pallas_call(
        body,
        out_shape=jax.ShapeDtypeStruct((out_rows, d), jnp.float32),
        in_specs=[pl.BlockSpec(memory_space=pltpu.VMEM)]
        + [pl.BlockSpec(memory_space=pl.ANY)] * 6,
        out_specs=pl.BlockSpec(memory_space=pltpu.VMEM),
        scratch_shapes=[
            pltpu.VMEM((b, d), jnp.bfloat16),
            pltpu.VMEM((N_LAYERS - 1, 3, b, d), jnp.bfloat16),
            pltpu.VMEM((3, out_rows, d), jnp.bfloat16),
            pltpu.VMEM((2, d_in, h_in), jnp.float32),
            pltpu.VMEM((2, h_in, d_in), jnp.float32),
            pltpu.VMEM((2, d_in, h_in), jnp.bfloat16),
            pltpu.VMEM((2, h_in, d_in), jnp.bfloat16),
            pltpu.SemaphoreType.DMA((N_LAYERS, CCHUNKS, 3)),
            pltpu.SemaphoreType.DMA((N_LAYERS, CCHUNKS, 3)),
            pltpu.SemaphoreType.DMA((2, WCHUNKS)),
            pltpu.SemaphoreType.DMA((2, WCHUNKS)),
        ],
        compiler_params=pltpu.CompilerParams(
            collective_id=0,
            vmem_limit_bytes=100 * 1024 * 1024,
        ),
    )(x, Win0, Wout0, Win1, Wout1, Win2, Wout2)


# device time: 29234 ns/iter; 1.0110x vs baseline; 1.0038x over previous
import jax
import jax.numpy as jnp
from jax import lax
from jax.experimental import pallas as pl
from jax.experimental.pallas import tpu as pltpu

N_DEV = 4
N_LAYERS = 3
WCHUNKS = 4
CCHUNKS = 2
FROM_LEFT, FROM_RIGHT, FROM_DIAG = 0, 1, 2


def kernel(x, Win0, Wout0, Win1, Wout1, Win2, Wout2):
    b, d = x.shape
    out_rows = b // N_DEV
    dc = d // CCHUNKS

    def body(x_ref, win0, wout0, win1, wout1, win2, wout2,
             out_ref, partial_ref, comm_ref, comm3_ref, win_buf, wout_buf,
             send_sems, recv_sems, win_dma_sems, wout_dma_sems):
        my = lax.axis_index("i")
        left = lax.rem(my + N_DEV - 1, N_DEV)
        right = lax.rem(my + 1, N_DEV)
        diag = lax.rem(my + 2, N_DEV)
        peers = ((left, FROM_RIGHT), (right, FROM_LEFT), (diag, FROM_DIAG))

        wins = [win0, win1, win2]
        wouts = [wout0, wout1, wout2]

        def start_weight_dma(l):
            win_copies, wout_copies = [], []
            for c in range(WCHUNKS):
                rw = pl.ds(c * (win_buf.shape[1] // WCHUNKS),
                           win_buf.shape[1] // WCHUNKS)
                cw = pltpu.make_async_copy(
                    wins[l].at[rw], win_buf.at[l % 2, rw],
                    win_dma_sems.at[l % 2, c])
                ro = pl.ds(c * (wout_buf.shape[1] // WCHUNKS),
                           wout_buf.shape[1] // WCHUNKS)
                co = pltpu.make_async_copy(
                    wouts[l].at[ro], wout_buf.at[l % 2, ro],
                    wout_dma_sems.at[l % 2, c])
                win_copies.append(cw)
                wout_copies.append(co)
            for cw in win_copies:
                cw.start()
            for co in wout_copies:
                co.start()
            return win_copies, wout_copies

        pending = start_weight_dma(0)

        barrier = pltpu.get_barrier_semaphore()
        for nbr, _ in peers:
            pl.semaphore_signal(
                barrier, inc=1,
                device_id=(nbr,), device_id_type=pl.DeviceIdType.MESH,
            )
        pl.semaphore_wait(barrier, 3)

        xb = x_ref[...].astype(jnp.bfloat16)
        for l in range(N_LAYERS):
            win_copies, wout_copies = pending
            for cw in win_copies:
                cw.wait()
            for co in wout_copies:
                co.wait()
            if l + 1 < N_LAYERS:
                pending = start_weight_dma(l + 1)

            h = jnp.dot(xb, win_buf[l % 2].astype(jnp.bfloat16),
                        preferred_element_type=jnp.float32)
            h = jnp.maximum(h, 0.0).astype(jnp.bfloat16)

            rdmas = []
            part_cols = []
            wout_bf = wout_buf[l % 2].astype(jnp.bfloat16)
            for c in range(CCHUNKS):
                cols = pl.ds(c * dc, dc)
                pc = jnp.dot(h, wout_bf[:, c * dc:(c + 1) * dc],
                             preferred_element_type=jnp.float32)
                part_cols.append(pc)
                partial_ref[:, cols] = pc.astype(jnp.bfloat16)
                chunk_rdmas = []
                for j, (peer, slot) in enumerate(peers):
                    if l < N_LAYERS - 1:
                        src = partial_ref.at[:, cols]
                        dst = comm_ref.at[l, slot, :, cols]
                    else:
                        src = partial_ref.at[
                            pl.ds(peer * out_rows, out_rows), cols]
                        dst = comm3_ref.at[slot, :, cols]
                    r = pltpu.make_async_remote_copy(
                        src_ref=src, dst_ref=dst,
                        send_sem=send_sems.at[l, c, j],
                        recv_sem=recv_sems.at[l, c, slot],
                        device_id=(peer,),
                        device_id_type=pl.DeviceIdType.MESH,
                    )
                    r.start()
                    chunk_rdmas.append(r)
                rdmas.append(chunk_rdmas)

            if l < N_LAYERS - 1:
                xb_cols = []
                for c in range(CCHUNKS):
                    for r in rdmas[c]:
                        r.wait_recv()
                    cols = slice(c * dc, (c + 1) * dc)
                    tot = (part_cols[c]
                           + comm_ref[l, FROM_LEFT, :, cols]
                           .astype(jnp.float32)
                           + comm_ref[l, FROM_RIGHT, :, cols]
                           .astype(jnp.float32)
                           + comm_ref[l, FROM_DIAG, :, cols]
                           .astype(jnp.float32))
                    xb_cols.append(tot.astype(jnp.bfloat16))
                xb = jnp.concatenate(xb_cols, axis=1)
            else:
                for chunk_rdmas in rdmas:
                    for r in chunk_rdmas:
                        r.wait_recv()
                rows = pl.ds(my * out_rows, out_rows)
                out_ref[...] = (
                    partial_ref[rows, :].astype(jnp.float32)
                    + comm3_ref[FROM_LEFT].astype(jnp.float32)
                    + comm3_ref[FROM_RIGHT].astype(jnp.float32)
                    + comm3_ref[FROM_DIAG].astype(jnp.float32))

            for chunk_rdmas in rdmas:
                for r in chunk_rdmas:
                    r.wait_send()

    d_in, h_in = Win0.shape
    return pl.pallas_call(
        body,
        out_shape=jax.ShapeDtypeStruct((out_rows, d), jnp.float32),
        in_specs=[pl.BlockSpec(memory_space=pltpu.VMEM)]
        + [pl.BlockSpec(memory_space=pl.ANY)] * 6,
        out_specs=pl.BlockSpec(memory_space=pltpu.VMEM),
        scratch_shapes=[
            pltpu.VMEM((b, d), jnp.bfloat16),
            pltpu.VMEM((N_LAYERS - 1, 3, b, d), jnp.bfloat16),
            pltpu.VMEM((3, out_rows, d), jnp.bfloat16),
            pltpu.VMEM((2, d_in, h_in), jnp.float32),
            pltpu.VMEM((2, h_in, d_in), jnp.float32),
            pltpu.SemaphoreType.DMA((N_LAYERS, CCHUNKS, 3)),
            pltpu.SemaphoreType.DMA((N_LAYERS, CCHUNKS, 3)),
            pltpu.SemaphoreType.DMA((2, WCHUNKS)),
            pltpu.SemaphoreType.DMA((2, WCHUNKS)),
        ],
        compiler_params=pltpu.CompilerParams(
            collective_id=0,
            vmem_limit_bytes=100 * 1024 * 1024,
        ),
    )(x, Win0, Wout0, Win1, Wout1, Win2, Wout2)


# device time: 28810 ns/iter; 1.0259x vs baseline; 1.0147x over previous
import jax
import jax.numpy as jnp
from jax import lax
from jax.experimental import pallas as pl
from jax.experimental.pallas import tpu as pltpu

N_DEV = 4
N_LAYERS = 3
WCHUNKS = 4
CCHUNKS = 2
FROM_LEFT, FROM_RIGHT, FROM_DIAG = 0, 1, 2


def kernel(x, Win0, Wout0, Win1, Wout1, Win2, Wout2):
    b, d = x.shape
    out_rows = b // N_DEV
    dc = d // CCHUNKS

    def body(x_ref, win0, wout0, win1, wout1, win2, wout2,
             out_ref, partial_ref, comm_ref, comm3_ref, win_buf, wout_buf,
             send_sems, recv_sems, win_dma_sems, wout_dma_sems):
        my = lax.axis_index("i")
        left = lax.rem(my + N_DEV - 1, N_DEV)
        right = lax.rem(my + 1, N_DEV)
        diag = lax.rem(my + 2, N_DEV)
        peers = ((left, FROM_RIGHT), (right, FROM_LEFT), (diag, FROM_DIAG))

        wins = [win0, win1, win2]
        wouts = [wout0, wout1, wout2]

        def start_weight_dma(l):
            win_copies, wout_copies = [], []
            for c in range(WCHUNKS):
                rw = pl.ds(c * (win_buf.shape[1] // WCHUNKS),
                           win_buf.shape[1] // WCHUNKS)
                cw = pltpu.make_async_copy(
                    wins[l].at[rw], win_buf.at[l % 2, rw],
                    win_dma_sems.at[l % 2, c])
                ro = pl.ds(c * (wout_buf.shape[1] // WCHUNKS),
                           wout_buf.shape[1] // WCHUNKS)
                co = pltpu.make_async_copy(
                    wouts[l].at[ro], wout_buf.at[l % 2, ro],
                    wout_dma_sems.at[l % 2, c])
                win_copies.append(cw)
                wout_copies.append(co)
            for cw in win_copies:
                cw.start()
            for co in wout_copies:
                co.start()
            return win_copies, wout_copies

        pending = start_weight_dma(0)

        barrier = pltpu.get_barrier_semaphore()
        for nbr, _ in peers:
            pl.semaphore_signal(
                barrier, inc=1,
                device_id=(nbr,), device_id_type=pl.DeviceIdType.MESH,
            )
        pl.semaphore_wait(barrier, 3)

        xb = x_ref[...].astype(jnp.bfloat16)
        for l in range(N_LAYERS):
            win_copies, wout_copies = pending
            if l == 0:
                kc = win_buf.shape[1] // WCHUNKS
                h32 = None
                for c in range(WCHUNKS):
                    win_copies[c].wait()
                    hc = jnp.dot(
                        xb[:, c * kc:(c + 1) * kc],
                        win_buf[l % 2, c * kc:(c + 1) * kc, :]
                        .astype(jnp.bfloat16),
                        preferred_element_type=jnp.float32)
                    h32 = hc if h32 is None else h32 + hc
            else:
                for cw in win_copies:
                    cw.wait()
                h32 = jnp.dot(xb, win_buf[l % 2].astype(jnp.bfloat16),
                              preferred_element_type=jnp.float32)
            for co in wout_copies:
                co.wait()
            if l + 1 < N_LAYERS:
                pending = start_weight_dma(l + 1)

            h = jnp.maximum(h32, 0.0).astype(jnp.bfloat16)

            rdmas = []
            part_cols = []
            wout_bf = wout_buf[l % 2].astype(jnp.bfloat16)
            for c in range(CCHUNKS):
                cols = pl.ds(c * dc, dc)
                pc = jnp.dot(h, wout_bf[:, c * dc:(c + 1) * dc],
                             preferred_element_type=jnp.float32)
                part_cols.append(pc)
                partial_ref[:, cols] = pc.astype(jnp.bfloat16)
                chunk_rdmas = []
                for j, (peer, slot) in enumerate(peers):
                    if l < N_LAYERS - 1:
                        src = partial_ref.at[:, cols]
                        dst = comm_ref.at[l, slot, :, cols]
                    else:
                        src = partial_ref.at[
                            pl.ds(peer * out_rows, out_rows), cols]
                        dst = comm3_ref.at[slot, :, cols]
                    r = pltpu.make_async_remote_copy(
                        src_ref=src, dst_ref=dst,
                        send_sem=send_sems.at[l, c, j],
                        recv_sem=recv_sems.at[l, c, slot],
                        device_id=(peer,),
                        device_id_type=pl.DeviceIdType.MESH,
                    )
                    r.start()
                    chunk_rdmas.append(r)
                rdmas.append(chunk_rdmas)

            if l < N_LAYERS - 1:
                xb_cols = []
                for c in range(CCHUNKS):
                    for r in rdmas[c]:
                        r.wait_recv()
                    cols = slice(c * dc, (c + 1) * dc)
                    tot = (part_cols[c]
                           + comm_ref[l, FROM_LEFT, :, cols]
                           .astype(jnp.float32)
                           + comm_ref[l, FROM_RIGHT, :, cols]
                           .astype(jnp.float32)
                           + comm_ref[l, FROM_DIAG, :, cols]
                           .astype(jnp.float32))
                    xb_cols.append(tot.astype(jnp.bfloat16))
                xb = jnp.concatenate(xb_cols, axis=1)
            else:
                for chunk_rdmas in rdmas:
                    for r in chunk_rdmas:
                        r.wait_recv()
                rows = pl.ds(my * out_rows, out_rows)
                out_ref[...] = (
                    partial_ref[rows, :].astype(jnp.float32)
                    + comm3_ref[FROM_LEFT].astype(jnp.float32)
                    + comm3_ref[FROM_RIGHT].astype(jnp.float32)
                    + comm3_ref[FROM_DIAG].astype(jnp.float32))

            for chunk_rdmas in rdmas:
                for r in chunk_rdmas:
                    r.wait_send()

    d_in, h_in = Win0.shape
    return pl.pallas_call(
        body,
        out_shape=jax.ShapeDtypeStruct((out_rows, d), jnp.float32),
        in_specs=[pl.BlockSpec(memory_space=pltpu.VMEM)]
        + [pl.BlockSpec(memory_space=pl.ANY)] * 6,
        out_specs=pl.BlockSpec(memory_space=pltpu.VMEM),
        scratch_shapes=[
            pltpu.VMEM((b, d), jnp.bfloat16),
            pltpu.VMEM((N_LAYERS - 1, 3, b, d), jnp.bfloat16),
            pltpu.VMEM((3, out_rows, d), jnp.bfloat16),
            pltpu.VMEM((2, d_in, h_in), jnp.float32),
            pltpu.VMEM((2, h_in, d_in), jnp.float32),
            pltpu.SemaphoreType.DMA((N_LAYERS, CCHUNKS, 3)),
            pltpu.SemaphoreType.DMA((N_LAYERS, CCHUNKS, 3)),
            pltpu.SemaphoreType.DMA((2, WCHUNKS)),
            pltpu.SemaphoreType.DMA((2, WCHUNKS)),
        ],
        compiler_params=pltpu.CompilerParams(
            collective_id=0,
            vmem_limit_bytes=100 * 1024 * 1024,
        ),
    )(x, Win0, Wout0, Win1, Wout1, Win2, Wout2)
